# Initial kernel scaffold; baseline (speedup 1.0000x reference)
import functools

import jax
import jax.numpy as jnp
from jax import lax
from jax.experimental import pallas as pl
from jax.experimental.pallas import tpu as pltpu
from jax.experimental.pallas import tpu_sc as plsc

_N = 99904
_E = 1598464
_HID = 64
_NSC = 2
_NTILE = 16
_B = 1024
_EP = 1605632
_NCHUNK = _EP // (_NSC * _NTILE * _B)
_EH = _EP // _NSC
_ET = _EH // _NTILE
_NT = _N // _NTILE
_NACC = _N + 16
_BN = 1784
_NBLK = _N // _BN


def _make_sc_prop(npass, gather):
    out_t = jax.ShapeDtypeStruct((_NSC, npass, _N, 16), jnp.float32)
    scratch = [
        pltpu.VMEM((_B // 128, 128), jnp.int32),
        pltpu.VMEM((_B,), jnp.int32),
        pltpu.VMEM((_B, 16), jnp.float32),
        pltpu.VMEM((_B, 16), jnp.float32),
        pltpu.VMEM_SHARED((_NACC, 16), jnp.float32),
        pltpu.SemaphoreType.DMA,
    ]
    mesh = plsc.VectorSubcoreMesh(core_axis_name="c", subcore_axis_name="s")

    @functools.partial(pl.kernel, out_type=out_t, mesh=mesh,
                       scratch_types=scratch)
    def k(*args):
        if gather:
            src_hbm, dst_hbm = args[0], args[1]
            gs = args[2:2 + npass]
            out_hbm = args[2 + npass]
            idx_d, idx_s, rows, zbuf, acc, sem = args[3 + npass:]
        else:
            dst_hbm = args[0]
            out_hbm = args[1]
            idx_d, idx_s, rows, zbuf, acc, sem = args[2:]
        c = lax.axis_index("c")
        s = lax.axis_index("s")
        ebase = c * _EH + s * _ET
        rowbase = (c * _EH + s * _ET) // 128

        @pl.loop(0, _B)
        def _zero(j):
            zbuf[j] = jnp.zeros((16,), jnp.float32)

        if not gather:
            @pl.loop(0, _B)
            def _ones(j):
                rows[j] = jnp.ones((16,), jnp.float32)

        off = s * _NT
        for p in range(npass):
            nmy = _NT + (_NACC - _N if p == 0 else 0)
            reps = (nmy + _B - 1) // _B
            for r in range(reps):
                sz = min(_B, nmy - r * _B)
                pltpu.sync_copy(zbuf.at[pl.ds(0, sz)],
                                acc.at[pl.ds(off + r * _B, sz)])
            plsc.subcore_barrier()

            @pl.loop(0, _NCHUNK)
            def _chunk(kk):
                base = ebase + kk * _B
                pltpu.sync_copy(dst_hbm.at[pl.ds(rowbase + kk * (_B // 128),
                                                 _B // 128)], idx_d)
                if gather:
                    pltpu.sync_copy(src_hbm.at[pl.ds(base, _B)], idx_s)
                    pltpu.async_copy(gs[p].at[idx_s], rows, sem).wait()
                for j in range(_B // 128):
                    pltpu.sync_copy(rows.at[pl.ds(j * 128, 128)],
                                    acc.at[idx_d.at[j]], add=True)

            plsc.subcore_barrier()
            pltpu.sync_copy(acc.at[pl.ds(off, _NT)],
                            out_hbm.at[c, p, pl.ds(off, _NT)])

    return k


_sc_deg = _make_sc_prop(1, gather=False)
_sc_prop1 = _make_sc_prop(1, gather=True)
_sc_prop4 = _make_sc_prop(4, gather=True)


def _prep_body(degp_ref, x_ref, xs_ref, dis_ref, dis2_ref):
    deg = degp_ref[0, 0, :, 0:1] + degp_ref[1, 0, :, 0:1] + 1.0
    dis = lax.rsqrt(deg)
    dis_ref[...] = dis
    dis2_ref[...] = dis * dis
    xs_ref[...] = jnp.concatenate(
        [x_ref[...] * dis, jnp.zeros((_BN, 13), jnp.float32)], axis=1)


def _prep_call(degp, x):
    return pl.pallas_call(
        _prep_body,
        grid=(_NBLK,),
        in_specs=[
            pl.BlockSpec((_NSC, 1, _BN, 16), lambda i: (0, 0, i, 0)),
            pl.BlockSpec((_BN, 3), lambda i: (i, 0)),
        ],
        out_specs=[
            pl.BlockSpec((_BN, 16), lambda i: (i, 0)),
            pl.BlockSpec((_BN, 1), lambda i: (i, 0)),
            pl.BlockSpec((_BN, 1), lambda i: (i, 0)),
        ],
        out_shape=[
            jax.ShapeDtypeStruct((_N, 16), jnp.float32),
            jax.ShapeDtypeStruct((_N, 1), jnp.float32),
            jax.ShapeDtypeStruct((_N, 1), jnp.float32),
        ],
    )(degp, x)


def _l1_body(p_ref, x_ref, dis_ref, dis2_ref, w1_ref, b1_ref,
             h1_ref, g0_ref, g1_ref, g2_ref, g3_ref):
    dis = dis_ref[...]
    q3 = dis * (p_ref[0, 0, :, 0:3] + p_ref[1, 0, :, 0:3]) \
        + dis2_ref[...] * x_ref[...]
    h1 = jnp.maximum(
        jnp.dot(q3, w1_ref[...], preferred_element_type=jnp.float32)
        + b1_ref[...], 0.0)
    h1_ref[...] = h1
    g = h1 * dis
    g0_ref[...] = g[:, 0:16]
    g1_ref[...] = g[:, 16:32]
    g2_ref[...] = g[:, 32:48]
    g3_ref[...] = g[:, 48:64]


def _l1_call(p1, x, dis, dis2, W1, b1):
    gspec = pl.BlockSpec((_BN, 16), lambda i: (i, 0))
    gshape = jax.ShapeDtypeStruct((_N, 16), jnp.float32)
    return pl.pallas_call(
        _l1_body,
        grid=(_NBLK,),
        in_specs=[
            pl.BlockSpec((_NSC, 1, _BN, 16), lambda i: (0, 0, i, 0)),
            pl.BlockSpec((_BN, 3), lambda i: (i, 0)),
            pl.BlockSpec((_BN, 1), lambda i: (i, 0)),
            pl.BlockSpec((_BN, 1), lambda i: (i, 0)),
            pl.BlockSpec((3, _HID), lambda i: (0, 0)),
            pl.BlockSpec((1, _HID), lambda i: (0, 0)),
        ],
        out_specs=[pl.BlockSpec((_BN, _HID), lambda i: (i, 0)),
                   gspec, gspec, gspec, gspec],
        out_shape=[jax.ShapeDtypeStruct((_N, _HID), jnp.float32),
                   gshape, gshape, gshape, gshape],
    )(p1, x, dis, dis2, W1, b1)


def _l2_body(q_ref, h1_ref, dis_ref, dis2_ref, w2_ref, b2_ref,
             wp_ref, bp_ref, wv_ref, bv_ref,
             logits_ref, msum_ref, v_ref):
    qs = q_ref[0] + q_ref[1]
    q64 = jnp.concatenate([qs[0], qs[1], qs[2], qs[3]], axis=1)
    a = dis_ref[...] * q64 + dis2_ref[...] * h1_ref[...]
    h2 = jnp.maximum(
        jnp.dot(a, w2_ref[...], preferred_element_type=jnp.float32)
        + b2_ref[...], 0.0)
    logits_ref[...] = (
        jnp.dot(h2, wp_ref[...], preferred_element_type=jnp.float32)
        + bp_ref[...])

    @pl.when(pl.program_id(0) == 0)
    def _():
        msum_ref[...] = jnp.zeros((1, _HID), jnp.float32)

    msum_ref[...] += jnp.sum(h2, axis=0, keepdims=True)

    @pl.when(pl.program_id(0) == _NBLK - 1)
    def _():
        m = msum_ref[...] * (1.0 / _N)
        v_ref[...] = jnp.tanh(
            jnp.dot(m, wv_ref[...], preferred_element_type=jnp.float32)
            + bv_ref[...])


def _l2_call(q, h1, dis, dis2, W2, b2, Wp, bp, Wv, bv):
    return pl.pallas_call(
        _l2_body,
        grid=(_NBLK,),
        in_specs=[
            pl.BlockSpec((_NSC, 4, _BN, 16), lambda i: (0, 0, i, 0)),
            pl.BlockSpec((_BN, _HID), lambda i: (i, 0)),
            pl.BlockSpec((_BN, 1), lambda i: (i, 0)),
            pl.BlockSpec((_BN, 1), lambda i: (i, 0)),
            pl.BlockSpec((_HID, _HID), lambda i: (0, 0)),
            pl.BlockSpec((1, _HID), lambda i: (0, 0)),
            pl.BlockSpec((_HID, 1), lambda i: (0, 0)),
            pl.BlockSpec((1, 1), lambda i: (0, 0)),
            pl.BlockSpec((_HID, 1), lambda i: (0, 0)),
            pl.BlockSpec((1, 1), lambda i: (0, 0)),
        ],
        out_specs=[
            pl.BlockSpec((_BN, 1), lambda i: (i, 0)),
            pl.BlockSpec((1, _HID), lambda i: (0, 0)),
            pl.BlockSpec((1, 1), lambda i: (0, 0)),
        ],
        out_shape=[
            jax.ShapeDtypeStruct((_N, 1), jnp.float32),
            jax.ShapeDtypeStruct((1, _HID), jnp.float32),
            jax.ShapeDtypeStruct((1, 1), jnp.float32),
        ],
    )(q, h1, dis, dis2, W2, b2, Wp, bp, Wv, bv)


@jax.jit
def kernel(x, edge_index, W1, b1, W2, b2, Wp, bp, Wv, bv):
    pad = _EP - _E
    src = jnp.concatenate([edge_index[0], jnp.zeros((pad,), jnp.int32)])
    dst = jnp.concatenate([edge_index[1], jnp.full((pad,), _N, jnp.int32)])
    dst2d = dst.reshape(_EP // 128, 128)

    degp = _sc_deg(dst2d)
    xs, dis, dis2 = _prep_call(degp, x)
    p1 = _sc_prop1(src, dst2d, xs)
    h1, g0, g1, g2, g3 = _l1_call(p1, x, dis, dis2, W1, b1.reshape(1, _HID))
    q = _sc_prop4(src, dst2d, g0, g1, g2, g3)
    logits, _msum, v = _l2_call(q, h1, dis, dis2, W2, b2.reshape(1, _HID),
                                Wp, bp.reshape(1, 1), Wv, bv.reshape(1, 1))
    return logits[:, 0], v[0]


# speedup vs baseline: 10.7095x; 10.7095x over previous
import functools

import jax
import jax.numpy as jnp
from jax import lax
from jax.experimental import pallas as pl
from jax.experimental.pallas import tpu as pltpu
from jax.experimental.pallas import tpu_sc as plsc

_N = 99904
_E = 1598464
_HID = 64
_NSC = 2
_NTILE = 16
_B = 512
_EP = 1605632
_NCHUNK = _EP // (_NSC * _NTILE * _B)
_EH = _EP // _NSC
_ET = _EH // _NTILE
_NF = 99968
_NT = _NF // _NTILE
_BN = 1784
_NBLK = _N // _BN


def _make_sc_prop(npass, gather):
    out_t = jax.ShapeDtypeStruct((_NSC, npass, _NF, 16), jnp.float32)
    scratch = [
        pltpu.VMEM((_B // 128, 128), jnp.int32),
        pltpu.VMEM((_B,), jnp.int32),
        pltpu.VMEM((_B, 16), jnp.float32),
        pltpu.VMEM((_B, 16), jnp.float32),
        pltpu.VMEM_SHARED((_NF, 16), jnp.float32),
        pltpu.SemaphoreType.DMA,
    ]
    mesh = plsc.VectorSubcoreMesh(core_axis_name="c", subcore_axis_name="s")

    @functools.partial(
        pl.kernel, out_type=out_t, mesh=mesh, scratch_types=scratch,
        compiler_params=pltpu.CompilerParams(use_tc_tiling_on_sc=False))
    def k(*args):
        if gather:
            src_hbm, dst_hbm = args[0], args[1]
            gs = args[2:2 + npass]
            out_hbm = args[2 + npass]
            idx_d, idx_s, rows, zbuf, acc, sem = args[3 + npass:]
        else:
            dst_hbm = args[0]
            out_hbm = args[1]
            idx_d, idx_s, rows, zbuf, acc, sem = args[2:]
        c = lax.axis_index("c")
        s = lax.axis_index("s")
        ebase = pl.multiple_of(c * _EH + s * _ET, 128)
        rowbase = pl.multiple_of((c * _EH + s * _ET) // 128, 8)

        @pl.loop(0, _B)
        def _zero(j):
            zbuf[j] = jnp.zeros((16,), jnp.float32)

        if not gather:
            @pl.loop(0, _B)
            def _ones(j):
                rows[j] = jnp.ones((16,), jnp.float32)

        off = pl.multiple_of(s * _NT, 8)
        for p in range(npass):
            reps = (_NT + _B - 1) // _B
            for r in range(reps):
                sz = min(_B, _NT - r * _B)
                pltpu.sync_copy(zbuf.at[pl.ds(0, sz)],
                                acc.at[pl.ds(off + r * _B, sz)])
            plsc.subcore_barrier()

            @pl.loop(0, _NCHUNK)
            def _chunk(kk):
                base = ebase + kk * _B
                pltpu.sync_copy(dst_hbm.at[pl.ds(rowbase + kk * (_B // 128),
                                                 _B // 128)], idx_d)
                if gather:
                    pltpu.sync_copy(src_hbm.at[pl.ds(base, _B)], idx_s)
                    pltpu.async_copy(gs[p].at[idx_s], rows, sem).wait()
                for j in range(_B // 128):
                    pltpu.sync_copy(rows.at[pl.ds(j * 128, 128)],
                                    acc.at[idx_d.at[j]], add=True)

            plsc.subcore_barrier()
            pltpu.sync_copy(acc.at[pl.ds(off, _NT)],
                            out_hbm.at[c, p, pl.ds(off, _NT)])

    return k


_sc_deg = _make_sc_prop(1, gather=False)
_sc_prop4 = _make_sc_prop(4, gather=True)


def _prep_body(degp_ref, x_ref, w1_ref, dis_ref, dis2_ref, h_ref,
               hs0_ref, hs1_ref, hs2_ref, hs3_ref):
    deg = degp_ref[0, 0, :, 0:1] + degp_ref[1, 0, :, 0:1] + 1.0
    dis = lax.rsqrt(deg)
    dis_ref[...] = dis
    dis2_ref[...] = dis * dis
    h = jnp.dot(x_ref[...], w1_ref[...], preferred_element_type=jnp.float32)
    h_ref[...] = h
    hs = h * dis
    hs0_ref[...] = hs[:, 0:16]
    hs1_ref[...] = hs[:, 16:32]
    hs2_ref[...] = hs[:, 32:48]
    hs3_ref[...] = hs[:, 48:64]


def _prep_call(degp, x, W1):
    gspec = pl.BlockSpec((_BN, 16), lambda i: (i, 0))
    gshape = jax.ShapeDtypeStruct((_N, 16), jnp.float32)
    return pl.pallas_call(
        _prep_body,
        grid=(_NBLK,),
        in_specs=[
            pl.BlockSpec((_NSC, 1, _BN, 16), lambda i: (0, 0, i, 0)),
            pl.BlockSpec((_BN, 3), lambda i: (i, 0)),
            pl.BlockSpec((3, _HID), lambda i: (0, 0)),
        ],
        out_specs=[
            pl.BlockSpec((_BN, 1), lambda i: (i, 0)),
            pl.BlockSpec((_BN, 1), lambda i: (i, 0)),
            pl.BlockSpec((_BN, _HID), lambda i: (i, 0)),
            gspec, gspec, gspec, gspec,
        ],
        out_shape=[
            jax.ShapeDtypeStruct((_N, 1), jnp.float32),
            jax.ShapeDtypeStruct((_N, 1), jnp.float32),
            jax.ShapeDtypeStruct((_N, _HID), jnp.float32),
            gshape, gshape, gshape, gshape,
        ],
    )(degp, x, W1)


def _mid_body(p_ref, h_ref, dis_ref, dis2_ref, b1_ref, w2_ref,
              y_ref, ys0_ref, ys1_ref, ys2_ref, ys3_ref):
    dis = dis_ref[...]
    ps = p_ref[0] + p_ref[1]
    p64 = jnp.concatenate([ps[0], ps[1], ps[2], ps[3]], axis=1)
    h1 = jnp.maximum(
        dis * p64 + dis2_ref[...] * h_ref[...] + b1_ref[...], 0.0)
    y = jnp.dot(h1, w2_ref[...], preferred_element_type=jnp.float32)
    y_ref[...] = y
    ys = y * dis
    ys0_ref[...] = ys[:, 0:16]
    ys1_ref[...] = ys[:, 16:32]
    ys2_ref[...] = ys[:, 32:48]
    ys3_ref[...] = ys[:, 48:64]


def _mid_call(p, h, dis, dis2, b1, W2):
    gspec = pl.BlockSpec((_BN, 16), lambda i: (i, 0))
    gshape = jax.ShapeDtypeStruct((_N, 16), jnp.float32)
    return pl.pallas_call(
        _mid_body,
        grid=(_NBLK,),
        in_specs=[
            pl.BlockSpec((_NSC, 4, _BN, 16), lambda i: (0, 0, i, 0)),
            pl.BlockSpec((_BN, _HID), lambda i: (i, 0)),
            pl.BlockSpec((_BN, 1), lambda i: (i, 0)),
            pl.BlockSpec((_BN, 1), lambda i: (i, 0)),
            pl.BlockSpec((1, _HID), lambda i: (0, 0)),
            pl.BlockSpec((_HID, _HID), lambda i: (0, 0)),
        ],
        out_specs=[pl.BlockSpec((_BN, _HID), lambda i: (i, 0)),
                   gspec, gspec, gspec, gspec],
        out_shape=[jax.ShapeDtypeStruct((_N, _HID), jnp.float32),
                   gshape, gshape, gshape, gshape],
    )(p, h, dis, dis2, b1, W2)


def _fin_body(q_ref, y_ref, dis_ref, dis2_ref, b2_ref, wp_ref, bp_ref,
              wv_ref, bv_ref, logits_ref, msum_ref, v_ref):
    qs = q_ref[0] + q_ref[1]
    q64 = jnp.concatenate([qs[0], qs[1], qs[2], qs[3]], axis=1)
    h2 = jnp.maximum(
        dis_ref[...] * q64 + dis2_ref[...] * y_ref[...] + b2_ref[...], 0.0)
    logits_ref[...] = (
        jnp.dot(h2, wp_ref[...], preferred_element_type=jnp.float32)
        + bp_ref[...])

    @pl.when(pl.program_id(0) == 0)
    def _():
        msum_ref[...] = jnp.zeros((1, _HID), jnp.float32)

    msum_ref[...] += jnp.sum(h2, axis=0, keepdims=True)

    @pl.when(pl.program_id(0) == _NBLK - 1)
    def _():
        m = msum_ref[...] * (1.0 / _N)
        v_ref[...] = jnp.tanh(
            jnp.dot(m, wv_ref[...], preferred_element_type=jnp.float32)
            + bv_ref[...])


def _fin_call(q, y, dis, dis2, b2, Wp, bp, Wv, bv):
    return pl.pallas_call(
        _fin_body,
        grid=(_NBLK,),
        in_specs=[
            pl.BlockSpec((_NSC, 4, _BN, 16), lambda i: (0, 0, i, 0)),
            pl.BlockSpec((_BN, _HID), lambda i: (i, 0)),
            pl.BlockSpec((_BN, 1), lambda i: (i, 0)),
            pl.BlockSpec((_BN, 1), lambda i: (i, 0)),
            pl.BlockSpec((1, _HID), lambda i: (0, 0)),
            pl.BlockSpec((_HID, 1), lambda i: (0, 0)),
            pl.BlockSpec((1, 1), lambda i: (0, 0)),
            pl.BlockSpec((_HID, 1), lambda i: (0, 0)),
            pl.BlockSpec((1, 1), lambda i: (0, 0)),
        ],
        out_specs=[
            pl.BlockSpec((_BN, 1), lambda i: (i, 0)),
            pl.BlockSpec((1, _HID), lambda i: (0, 0)),
            pl.BlockSpec((1, 1), lambda i: (0, 0)),
        ],
        out_shape=[
            jax.ShapeDtypeStruct((_N, 1), jnp.float32),
            jax.ShapeDtypeStruct((1, _HID), jnp.float32),
            jax.ShapeDtypeStruct((1, 1), jnp.float32),
        ],
    )(q, y, dis, dis2, b2, Wp, bp, Wv, bv)


@jax.jit
def kernel(x, edge_index, W1, b1, W2, b2, Wp, bp, Wv, bv):
    pad = _EP - _E
    src = jnp.concatenate([edge_index[0], jnp.zeros((pad,), jnp.int32)])
    dst = jnp.concatenate([edge_index[1], jnp.full((pad,), _N, jnp.int32)])
    dst2d = dst.reshape(_EP // 128, 128)

    degp = _sc_deg(dst2d)
    dis, dis2, h, hs0, hs1, hs2, hs3 = _prep_call(degp, x, W1)
    p = _sc_prop4(src, dst2d, hs0, hs1, hs2, hs3)
    y, ys0, ys1, ys2, ys3 = _mid_call(p, h, dis, dis2,
                                      b1.reshape(1, _HID), W2)
    q = _sc_prop4(src, dst2d, ys0, ys1, ys2, ys3)
    logits, _msum, v = _fin_call(q, y, dis, dis2, b2.reshape(1, _HID),
                                 Wp, bp.reshape(1, 1), Wv, bv.reshape(1, 1))
    return logits[:, 0], v[0]


# speedup vs baseline: 14.6507x; 1.3680x over previous
import functools

import jax
import jax.numpy as jnp
from jax import lax
from jax.experimental import pallas as pl
from jax.experimental.pallas import tpu as pltpu
from jax.experimental.pallas import tpu_sc as plsc

_N = 99904
_E = 1598464
_HID = 64
_NSC = 2
_NTILE = 16
_B = 512
_EP = 1605632
_NCHUNK = _EP // (_NSC * _NTILE * _B)
_EH = _EP // _NSC
_ET = _EH // _NTILE
_NF = 99968
_NT = _NF // _NTILE
_BN = 1784
_NBLK = _N // _BN


def _make_sc_prop(npass, gather):
    out_t = jax.ShapeDtypeStruct((_NSC, npass, _NF, 16), jnp.float32)
    _RB = _B // 128
    scratch = [
        pltpu.VMEM((_RB, 128), jnp.int32),
        pltpu.VMEM((_RB, 128), jnp.int32),
        pltpu.VMEM((_B,), jnp.int32),
        pltpu.VMEM((_B,), jnp.int32),
        pltpu.VMEM((_B, 16), jnp.float32),
        pltpu.VMEM((_B, 16), jnp.float32),
        pltpu.VMEM_SHARED((_NF, 16), jnp.float32),
        pltpu.SemaphoreType.DMA,
        pltpu.SemaphoreType.DMA,
        pltpu.SemaphoreType.DMA,
        pltpu.SemaphoreType.DMA,
        pltpu.SemaphoreType.DMA,
    ]
    mesh = plsc.VectorSubcoreMesh(core_axis_name="c", subcore_axis_name="s")

    @functools.partial(
        pl.kernel, out_type=out_t, mesh=mesh, scratch_types=scratch,
        compiler_params=pltpu.CompilerParams(use_tc_tiling_on_sc=False))
    def k(*args):
        if gather:
            src_hbm, dst_hbm = args[0], args[1]
            gs = args[2:2 + npass]
            out_hbm = args[2 + npass]
            rest = args[3 + npass:]
        else:
            dst_hbm = args[0]
            out_hbm = args[1]
            rest = args[2:]
        (idx_d0, idx_d1, idx_s0, idx_s1, rows0, rows1, acc,
         si0, si1, sg0, sg1, ss) = rest
        idx_d = (idx_d0, idx_d1)
        idx_s = (idx_s0, idx_s1)
        rows = (rows0, rows1)
        si = (si0, si1)
        sg = (sg0, sg1)
        c = lax.axis_index("c")
        s = lax.axis_index("s")
        ebase = pl.multiple_of(c * _EH + s * _ET, 128)
        rowbase = pl.multiple_of((c * _EH + s * _ET) // 128, 8)
        npair = _NCHUNK // 2

        def idx_issue(kk, b):
            pltpu.async_copy(
                dst_hbm.at[pl.ds(rowbase + kk * _RB, _RB)], idx_d[b], si[b])
            if gather:
                pltpu.async_copy(
                    src_hbm.at[pl.ds(ebase + kk * _B, _B)], idx_s[b], si[b])

        def idx_wait(b):
            pltpu.make_async_copy(
                dst_hbm.at[pl.ds(rowbase, _RB)], idx_d[b], si[b]).wait()
            if gather:
                pltpu.make_async_copy(
                    src_hbm.at[pl.ds(ebase, _B)], idx_s[b], si[b]).wait()

        def scat_issue(b):
            for j in range(_RB):
                pltpu.async_copy(rows[b].at[pl.ds(j * 128, 128)],
                                 acc.at[idx_d[b].at[j]], ss, add=True)

        def scat_wait():
            for _ in range(2 * _RB):
                pltpu.make_async_copy(rows0.at[pl.ds(0, 128)],
                                      acc.at[idx_d0.at[0]], ss).wait()

        off = pl.multiple_of(s * _NT, 8)
        for p in range(npass):
            @pl.loop(0, _B)
            def _zero(j):
                rows0[j] = jnp.zeros((16,), jnp.float32)

            reps = (_NT + _B - 1) // _B
            for r in range(reps):
                sz = min(_B, _NT - r * _B)
                pltpu.sync_copy(rows0.at[pl.ds(0, sz)],
                                acc.at[pl.ds(off + r * _B, sz)])
            if not gather:
                @pl.loop(0, _B)
                def _ones(j):
                    rows0[j] = jnp.ones((16,), jnp.float32)
                    rows1[j] = jnp.ones((16,), jnp.float32)
            plsc.subcore_barrier()

            idx_issue(0, 0)
            idx_issue(1, 1)

            @pl.loop(0, npair)
            def _pair(t):
                a = t * 2
                idx_wait(0)
                if gather:
                    pltpu.async_copy(gs[p].at[idx_s0], rows0, sg0)
                idx_wait(1)
                if gather:
                    pltpu.async_copy(gs[p].at[idx_s1], rows1, sg1)
                    pltpu.make_async_copy(gs[p].at[idx_s0], rows0, sg0).wait()
                scat_issue(0)
                if gather:
                    pltpu.make_async_copy(gs[p].at[idx_s1], rows1, sg1).wait()
                scat_issue(1)
                scat_wait()

                @pl.when(t < npair - 1)
                def _():
                    idx_issue(a + 2, 0)
                    idx_issue(a + 3, 1)

            plsc.subcore_barrier()
            pltpu.sync_copy(acc.at[pl.ds(off, _NT)],
                            out_hbm.at[c, p, pl.ds(off, _NT)])

    return k


_sc_deg = _make_sc_prop(1, gather=False)
_sc_prop4 = _make_sc_prop(4, gather=True)


def _prep_body(degp_ref, x_ref, w1_ref, dis_ref, dis2_ref, h_ref,
               hs0_ref, hs1_ref, hs2_ref, hs3_ref):
    deg = degp_ref[0, 0, :, 0:1] + degp_ref[1, 0, :, 0:1] + 1.0
    dis = lax.rsqrt(deg)
    dis_ref[...] = dis
    dis2_ref[...] = dis * dis
    h = jnp.dot(x_ref[...], w1_ref[...], preferred_element_type=jnp.float32)
    h_ref[...] = h
    hs = h * dis
    hs0_ref[...] = hs[:, 0:16]
    hs1_ref[...] = hs[:, 16:32]
    hs2_ref[...] = hs[:, 32:48]
    hs3_ref[...] = hs[:, 48:64]


def _prep_call(degp, x, W1):
    gspec = pl.BlockSpec((_BN, 16), lambda i: (i, 0))
    gshape = jax.ShapeDtypeStruct((_N, 16), jnp.float32)
    return pl.pallas_call(
        _prep_body,
        grid=(_NBLK,),
        in_specs=[
            pl.BlockSpec((_NSC, 1, _BN, 16), lambda i: (0, 0, i, 0)),
            pl.BlockSpec((_BN, 3), lambda i: (i, 0)),
            pl.BlockSpec((3, _HID), lambda i: (0, 0)),
        ],
        out_specs=[
            pl.BlockSpec((_BN, 1), lambda i: (i, 0)),
            pl.BlockSpec((_BN, 1), lambda i: (i, 0)),
            pl.BlockSpec((_BN, _HID), lambda i: (i, 0)),
            gspec, gspec, gspec, gspec,
        ],
        out_shape=[
            jax.ShapeDtypeStruct((_N, 1), jnp.float32),
            jax.ShapeDtypeStruct((_N, 1), jnp.float32),
            jax.ShapeDtypeStruct((_N, _HID), jnp.float32),
            gshape, gshape, gshape, gshape,
        ],
    )(degp, x, W1)


def _mid_body(p_ref, h_ref, dis_ref, dis2_ref, b1_ref, w2_ref,
              y_ref, ys0_ref, ys1_ref, ys2_ref, ys3_ref):
    dis = dis_ref[...]
    ps = p_ref[0] + p_ref[1]
    p64 = jnp.concatenate([ps[0], ps[1], ps[2], ps[3]], axis=1)
    h1 = jnp.maximum(
        dis * p64 + dis2_ref[...] * h_ref[...] + b1_ref[...], 0.0)
    y = jnp.dot(h1, w2_ref[...], preferred_element_type=jnp.float32)
    y_ref[...] = y
    ys = y * dis
    ys0_ref[...] = ys[:, 0:16]
    ys1_ref[...] = ys[:, 16:32]
    ys2_ref[...] = ys[:, 32:48]
    ys3_ref[...] = ys[:, 48:64]


def _mid_call(p, h, dis, dis2, b1, W2):
    gspec = pl.BlockSpec((_BN, 16), lambda i: (i, 0))
    gshape = jax.ShapeDtypeStruct((_N, 16), jnp.float32)
    return pl.pallas_call(
        _mid_body,
        grid=(_NBLK,),
        in_specs=[
            pl.BlockSpec((_NSC, 4, _BN, 16), lambda i: (0, 0, i, 0)),
            pl.BlockSpec((_BN, _HID), lambda i: (i, 0)),
            pl.BlockSpec((_BN, 1), lambda i: (i, 0)),
            pl.BlockSpec((_BN, 1), lambda i: (i, 0)),
            pl.BlockSpec((1, _HID), lambda i: (0, 0)),
            pl.BlockSpec((_HID, _HID), lambda i: (0, 0)),
        ],
        out_specs=[pl.BlockSpec((_BN, _HID), lambda i: (i, 0)),
                   gspec, gspec, gspec, gspec],
        out_shape=[jax.ShapeDtypeStruct((_N, _HID), jnp.float32),
                   gshape, gshape, gshape, gshape],
    )(p, h, dis, dis2, b1, W2)


def _fin_body(q_ref, y_ref, dis_ref, dis2_ref, b2_ref, wp_ref, bp_ref,
              wv_ref, bv_ref, logits_ref, msum_ref, v_ref):
    qs = q_ref[0] + q_ref[1]
    q64 = jnp.concatenate([qs[0], qs[1], qs[2], qs[3]], axis=1)
    h2 = jnp.maximum(
        dis_ref[...] * q64 + dis2_ref[...] * y_ref[...] + b2_ref[...], 0.0)
    logits_ref[...] = (
        jnp.dot(h2, wp_ref[...], preferred_element_type=jnp.float32)
        + bp_ref[...])

    @pl.when(pl.program_id(0) == 0)
    def _():
        msum_ref[...] = jnp.zeros((1, _HID), jnp.float32)

    msum_ref[...] += jnp.sum(h2, axis=0, keepdims=True)

    @pl.when(pl.program_id(0) == _NBLK - 1)
    def _():
        m = msum_ref[...] * (1.0 / _N)
        v_ref[...] = jnp.tanh(
            jnp.dot(m, wv_ref[...], preferred_element_type=jnp.float32)
            + bv_ref[...])


def _fin_call(q, y, dis, dis2, b2, Wp, bp, Wv, bv):
    return pl.pallas_call(
        _fin_body,
        grid=(_NBLK,),
        in_specs=[
            pl.BlockSpec((_NSC, 4, _BN, 16), lambda i: (0, 0, i, 0)),
            pl.BlockSpec((_BN, _HID), lambda i: (i, 0)),
            pl.BlockSpec((_BN, 1), lambda i: (i, 0)),
            pl.BlockSpec((_BN, 1), lambda i: (i, 0)),
            pl.BlockSpec((1, _HID), lambda i: (0, 0)),
            pl.BlockSpec((_HID, 1), lambda i: (0, 0)),
            pl.BlockSpec((1, 1), lambda i: (0, 0)),
            pl.BlockSpec((_HID, 1), lambda i: (0, 0)),
            pl.BlockSpec((1, 1), lambda i: (0, 0)),
        ],
        out_specs=[
            pl.BlockSpec((_BN, 1), lambda i: (i, 0)),
            pl.BlockSpec((1, _HID), lambda i: (0, 0)),
            pl.BlockSpec((1, 1), lambda i: (0, 0)),
        ],
        out_shape=[
            jax.ShapeDtypeStruct((_N, 1), jnp.float32),
            jax.ShapeDtypeStruct((1, _HID), jnp.float32),
            jax.ShapeDtypeStruct((1, 1), jnp.float32),
        ],
    )(q, y, dis, dis2, b2, Wp, bp, Wv, bv)


@jax.jit
def kernel(x, edge_index, W1, b1, W2, b2, Wp, bp, Wv, bv):
    pad = _EP - _E
    src = jnp.concatenate([edge_index[0], jnp.zeros((pad,), jnp.int32)])
    dst = jnp.concatenate([edge_index[1], jnp.full((pad,), _N, jnp.int32)])
    dst2d = dst.reshape(_EP // 128, 128)

    degp = _sc_deg(dst2d)
    dis, dis2, h, hs0, hs1, hs2, hs3 = _prep_call(degp, x, W1)
    p = _sc_prop4(src, dst2d, hs0, hs1, hs2, hs3)
    y, ys0, ys1, ys2, ys3 = _mid_call(p, h, dis, dis2,
                                      b1.reshape(1, _HID), W2)
    q = _sc_prop4(src, dst2d, ys0, ys1, ys2, ys3)
    logits, _msum, v = _fin_call(q, y, dis, dis2, b2.reshape(1, _HID),
                                 Wp, bp.reshape(1, 1), Wv, bv.reshape(1, 1))
    return logits[:, 0], v[0]


# speedup vs baseline: 15.9010x; 1.0853x over previous
import functools

import jax
import jax.numpy as jnp
from jax import lax
from jax.experimental import pallas as pl
from jax.experimental.pallas import tpu as pltpu
from jax.experimental.pallas import tpu_sc as plsc

_N = 99904
_E = 1598464
_HID = 64
_NSC = 2
_NTILE = 16
_B = 256
_NBUF = 4
_EP = 1605632
_NCHUNK = _EP // (_NSC * _NTILE * _B)
_EH = _EP // _NSC
_ET = _EH // _NTILE
_NF = 99968
_NT = _NF // _NTILE
_BN = 1784
_NBLK = _N // _BN


def _make_sc_prop(npass, gather):
    out_t = jax.ShapeDtypeStruct((_NSC, npass, _NF, 16), jnp.float32)
    _GB = _B * _NBUF
    _GR = _GB // 128
    _RB = _B // 128
    scratch = (
        [pltpu.VMEM((_GR, 128), jnp.int32) for _ in range(2)]
        + [pltpu.VMEM((_GB,), jnp.int32) for _ in range(2)]
        + [pltpu.VMEM((_B, 16), jnp.float32) for _ in range(_NBUF)]
        + [pltpu.VMEM_SHARED((_NF, 16), jnp.float32)]
        + [pltpu.SemaphoreType.DMA for _ in range(2)]
        + [pltpu.SemaphoreType.DMA for _ in range(_NBUF)]
        + [pltpu.SemaphoreType.DMA]
    )
    mesh = plsc.VectorSubcoreMesh(core_axis_name="c", subcore_axis_name="s")

    @functools.partial(
        pl.kernel, out_type=out_t, mesh=mesh, scratch_types=scratch,
        compiler_params=pltpu.CompilerParams(use_tc_tiling_on_sc=False))
    def k(*args):
        if gather:
            src_hbm, dst_hbm = args[0], args[1]
            gs = args[2:2 + npass]
            out_hbm = args[2 + npass]
            rest = args[3 + npass:]
        else:
            dst_hbm = args[0]
            out_hbm = args[1]
            rest = args[2:]
        idx_d = rest[0:2]
        idx_s = rest[2:4]
        rows = rest[4:4 + _NBUF]
        acc = rest[4 + _NBUF]
        si = rest[5 + _NBUF:7 + _NBUF]
        sg = rest[7 + _NBUF:7 + 2 * _NBUF]
        ss = rest[7 + 2 * _NBUF]
        c = lax.axis_index("c")
        s = lax.axis_index("s")
        ebase = pl.multiple_of(c * _EH + s * _ET, 128)
        rowbase = pl.multiple_of((c * _EH + s * _ET) // 128, 8)
        ngrp = _NCHUNK // _NBUF

        def idx_issue(g, u):
            pltpu.async_copy(
                dst_hbm.at[pl.ds(rowbase + g * _GR, _GR)], idx_d[u], si[u])
            if gather:
                pltpu.async_copy(
                    src_hbm.at[pl.ds(ebase + g * _GB, _GB)], idx_s[u], si[u])

        def idx_wait(u):
            pltpu.make_async_copy(
                dst_hbm.at[pl.ds(rowbase, _GR)], idx_d[u], si[u]).wait()
            if gather:
                pltpu.make_async_copy(
                    src_hbm.at[pl.ds(ebase, _GB)], idx_s[u], si[u]).wait()

        def proc_group(g, u, prefetch):
            idx_wait(u)
            if prefetch:
                @pl.when(g + 1 < ngrp)
                def _():
                    idx_issue(g + 1, 1 - u)
            if gather:
                for b in range(_NBUF):
                    pltpu.async_copy(
                        gs[p_cur[0]].at[idx_s[u].at[pl.ds(b * _B, _B)]],
                        rows[b], sg[b])
                for b in range(_NBUF):
                    pltpu.make_async_copy(
                        gs[p_cur[0]].at[idx_s[u].at[pl.ds(0, _B)]],
                        rows[b], sg[b]).wait()
                    for j in range(_RB):
                        pltpu.async_copy(
                            rows[b].at[pl.ds(j * 128, 128)],
                            acc.at[idx_d[u].at[b * _RB + j]], ss, add=True)
            else:
                for b in range(_NBUF):
                    for j in range(_RB):
                        pltpu.async_copy(
                            rows[b].at[pl.ds(j * 128, 128)],
                            acc.at[idx_d[u].at[b * _RB + j]], ss, add=True)
            for _ in range(_NBUF * _RB):
                pltpu.make_async_copy(rows[0].at[pl.ds(0, 128)],
                                      acc.at[idx_d[0].at[0]], ss).wait()

        p_cur = [0]
        off = pl.multiple_of(s * _NT, 8)
        for p in range(npass):
            p_cur[0] = p
            @pl.loop(0, _B)
            def _zero(j):
                rows[0][j] = jnp.zeros((16,), jnp.float32)

            reps = (_NT + _B - 1) // _B
            for r in range(reps):
                sz = min(_B, _NT - r * _B)
                pltpu.sync_copy(rows[0].at[pl.ds(0, sz)],
                                acc.at[pl.ds(off + r * _B, sz)])
            if not gather:
                @pl.loop(0, _B)
                def _ones(j):
                    for b in range(_NBUF):
                        rows[b][j] = jnp.ones((16,), jnp.float32)
            plsc.subcore_barrier()

            idx_issue(0, 0)

            @pl.loop(0, ngrp // 2)
            def _pair(v):
                proc_group(2 * v, 0, True)
                proc_group(2 * v + 1, 1, True)

            if ngrp % 2 == 1:
                proc_group(ngrp - 1, 0, False)

            plsc.subcore_barrier()
            pltpu.sync_copy(acc.at[pl.ds(off, _NT)],
                            out_hbm.at[c, p, pl.ds(off, _NT)])

    return k


_sc_deg = _make_sc_prop(1, gather=False)
_sc_prop4 = _make_sc_prop(4, gather=True)


def _prep_body(degp_ref, x_ref, w1_ref, dis_ref, dis2_ref, h_ref,
               hs0_ref, hs1_ref, hs2_ref, hs3_ref):
    deg = degp_ref[0, 0, :, 0:1] + degp_ref[1, 0, :, 0:1] + 1.0
    dis = lax.rsqrt(deg)
    dis_ref[...] = dis
    dis2_ref[...] = dis * dis
    h = jnp.dot(x_ref[...], w1_ref[...], preferred_element_type=jnp.float32)
    h_ref[...] = h
    hs = h * dis
    hs0_ref[...] = hs[:, 0:16]
    hs1_ref[...] = hs[:, 16:32]
    hs2_ref[...] = hs[:, 32:48]
    hs3_ref[...] = hs[:, 48:64]


def _prep_call(degp, x, W1):
    gspec = pl.BlockSpec((_BN, 16), lambda i: (i, 0))
    gshape = jax.ShapeDtypeStruct((_N, 16), jnp.float32)
    return pl.pallas_call(
        _prep_body,
        grid=(_NBLK,),
        in_specs=[
            pl.BlockSpec((_NSC, 1, _BN, 16), lambda i: (0, 0, i, 0)),
            pl.BlockSpec((_BN, 3), lambda i: (i, 0)),
            pl.BlockSpec((3, _HID), lambda i: (0, 0)),
        ],
        out_specs=[
            pl.BlockSpec((_BN, 1), lambda i: (i, 0)),
            pl.BlockSpec((_BN, 1), lambda i: (i, 0)),
            pl.BlockSpec((_BN, _HID), lambda i: (i, 0)),
            gspec, gspec, gspec, gspec,
        ],
        out_shape=[
            jax.ShapeDtypeStruct((_N, 1), jnp.float32),
            jax.ShapeDtypeStruct((_N, 1), jnp.float32),
            jax.ShapeDtypeStruct((_N, _HID), jnp.float32),
            gshape, gshape, gshape, gshape,
        ],
    )(degp, x, W1)


def _mid_body(p_ref, h_ref, dis_ref, dis2_ref, b1_ref, w2_ref,
              y_ref, ys0_ref, ys1_ref, ys2_ref, ys3_ref):
    dis = dis_ref[...]
    ps = p_ref[0] + p_ref[1]
    p64 = jnp.concatenate([ps[0], ps[1], ps[2], ps[3]], axis=1)
    h1 = jnp.maximum(
        dis * p64 + dis2_ref[...] * h_ref[...] + b1_ref[...], 0.0)
    y = jnp.dot(h1, w2_ref[...], preferred_element_type=jnp.float32)
    y_ref[...] = y
    ys = y * dis
    ys0_ref[...] = ys[:, 0:16]
    ys1_ref[...] = ys[:, 16:32]
    ys2_ref[...] = ys[:, 32:48]
    ys3_ref[...] = ys[:, 48:64]


def _mid_call(p, h, dis, dis2, b1, W2):
    gspec = pl.BlockSpec((_BN, 16), lambda i: (i, 0))
    gshape = jax.ShapeDtypeStruct((_N, 16), jnp.float32)
    return pl.pallas_call(
        _mid_body,
        grid=(_NBLK,),
        in_specs=[
            pl.BlockSpec((_NSC, 4, _BN, 16), lambda i: (0, 0, i, 0)),
            pl.BlockSpec((_BN, _HID), lambda i: (i, 0)),
            pl.BlockSpec((_BN, 1), lambda i: (i, 0)),
            pl.BlockSpec((_BN, 1), lambda i: (i, 0)),
            pl.BlockSpec((1, _HID), lambda i: (0, 0)),
            pl.BlockSpec((_HID, _HID), lambda i: (0, 0)),
        ],
        out_specs=[pl.BlockSpec((_BN, _HID), lambda i: (i, 0)),
                   gspec, gspec, gspec, gspec],
        out_shape=[jax.ShapeDtypeStruct((_N, _HID), jnp.float32),
                   gshape, gshape, gshape, gshape],
    )(p, h, dis, dis2, b1, W2)


def _fin_body(q_ref, y_ref, dis_ref, dis2_ref, b2_ref, wp_ref, bp_ref,
              wv_ref, bv_ref, logits_ref, msum_ref, v_ref):
    qs = q_ref[0] + q_ref[1]
    q64 = jnp.concatenate([qs[0], qs[1], qs[2], qs[3]], axis=1)
    h2 = jnp.maximum(
        dis_ref[...] * q64 + dis2_ref[...] * y_ref[...] + b2_ref[...], 0.0)
    logits_ref[...] = (
        jnp.dot(h2, wp_ref[...], preferred_element_type=jnp.float32)
        + bp_ref[...])

    @pl.when(pl.program_id(0) == 0)
    def _():
        msum_ref[...] = jnp.zeros((1, _HID), jnp.float32)

    msum_ref[...] += jnp.sum(h2, axis=0, keepdims=True)

    @pl.when(pl.program_id(0) == _NBLK - 1)
    def _():
        m = msum_ref[...] * (1.0 / _N)
        v_ref[...] = jnp.tanh(
            jnp.dot(m, wv_ref[...], preferred_element_type=jnp.float32)
            + bv_ref[...])


def _fin_call(q, y, dis, dis2, b2, Wp, bp, Wv, bv):
    return pl.pallas_call(
        _fin_body,
        grid=(_NBLK,),
        in_specs=[
            pl.BlockSpec((_NSC, 4, _BN, 16), lambda i: (0, 0, i, 0)),
            pl.BlockSpec((_BN, _HID), lambda i: (i, 0)),
            pl.BlockSpec((_BN, 1), lambda i: (i, 0)),
            pl.BlockSpec((_BN, 1), lambda i: (i, 0)),
            pl.BlockSpec((1, _HID), lambda i: (0, 0)),
            pl.BlockSpec((_HID, 1), lambda i: (0, 0)),
            pl.BlockSpec((1, 1), lambda i: (0, 0)),
            pl.BlockSpec((_HID, 1), lambda i: (0, 0)),
            pl.BlockSpec((1, 1), lambda i: (0, 0)),
        ],
        out_specs=[
            pl.BlockSpec((_BN, 1), lambda i: (i, 0)),
            pl.BlockSpec((1, _HID), lambda i: (0, 0)),
            pl.BlockSpec((1, 1), lambda i: (0, 0)),
        ],
        out_shape=[
            jax.ShapeDtypeStruct((_N, 1), jnp.float32),
            jax.ShapeDtypeStruct((1, _HID), jnp.float32),
            jax.ShapeDtypeStruct((1, 1), jnp.float32),
        ],
    )(q, y, dis, dis2, b2, Wp, bp, Wv, bv)


@jax.jit
def kernel(x, edge_index, W1, b1, W2, b2, Wp, bp, Wv, bv):
    pad = _EP - _E
    src = jnp.concatenate([edge_index[0], jnp.zeros((pad,), jnp.int32)])
    dst = jnp.concatenate([edge_index[1], jnp.full((pad,), _N, jnp.int32)])
    dst2d = dst.reshape(_EP // 128, 128)

    degp = _sc_deg(dst2d)
    dis, dis2, h, hs0, hs1, hs2, hs3 = _prep_call(degp, x, W1)
    p = _sc_prop4(src, dst2d, hs0, hs1, hs2, hs3)
    y, ys0, ys1, ys2, ys3 = _mid_call(p, h, dis, dis2,
                                      b1.reshape(1, _HID), W2)
    q = _sc_prop4(src, dst2d, ys0, ys1, ys2, ys3)
    logits, _msum, v = _fin_call(q, y, dis, dis2, b2.reshape(1, _HID),
                                 Wp, bp.reshape(1, 1), Wv, bv.reshape(1, 1))
    return logits[:, 0], v[0]
